# Initial kernel scaffold; baseline (speedup 1.0000x reference)
import jax
import jax.numpy as jnp
from jax.experimental import pallas as pl


def kernel(x, w1r, s1, t1, w2r, s2, t2, fw1p, s3, t3, fw2t, fb2r):
    raise NotImplementedError("write your pallas kernel here")


# speedup vs baseline: 16.7307x; 16.7307x over previous
import functools

import jax
import jax.numpy as jnp
from jax.experimental import pallas as pl
from jax.experimental.pallas import tpu as pltpu


def _fused_net_kernel(x_ref, w1_ref, t1_ref, w2_ref, s2_ref, t2_ref,
                      fw1_ref, s3_ref, t3_ref, fw2_ref, fb2_ref, o_ref, *, bt):
    x = x_ref[...]
    wins = [x[:, ki:ki + 24, :].reshape(bt * 24, 28) for ki in range(5)]
    p = jnp.concatenate(wins, axis=1)
    t1 = t1_ref[...]
    h = jnp.dot(p, w1_ref[0], preferred_element_type=jnp.float32) + t1
    for j in (1, 2):
        h = jnp.maximum(
            h, jnp.dot(p, w1_ref[j], preferred_element_type=jnp.float32) + t1)
    h = jnp.maximum(h, 0.0).astype(jnp.bfloat16)
    y1 = jnp.max(h.reshape(bt * 8, 3, 160), axis=1)
    y1 = y1.reshape(bt, 8, 160)

    acc = jnp.zeros((bt * 4, 200), jnp.float32)
    for ki in range(5):
        q = y1[:, ki:ki + 4, :].reshape(bt * 4, 160)
        acc = acc + jnp.dot(q, w2_ref[ki], preferred_element_type=jnp.float32)
    z = jnp.maximum(acc * s2_ref[...] + t2_ref[...], 0.0)
    z = jnp.concatenate(
        [jnp.maximum(z[:, 0:50], z[:, 50:100]),
         jnp.maximum(z[:, 100:150], z[:, 150:200])], axis=1)
    z = jnp.max(z.reshape(bt * 2, 2, 100), axis=1)
    z = z.reshape(bt, 2, 100)
    flat = jnp.concatenate([z[:, 0, :], z[:, 1, :]], axis=1)
    flat = flat.astype(jnp.bfloat16)

    g = jnp.dot(flat, fw1_ref[...], preferred_element_type=jnp.float32)
    g = jnp.maximum(g * s3_ref[...] + t3_ref[...], 0.0).astype(jnp.bfloat16)
    o = jnp.dot(g, fw2_ref[...], preferred_element_type=jnp.float32)
    o_ref[...] = o + fb2_ref[...]


def _band_conv1(w1r, s1):
    w = w1r.reshape(5, 5, 20) * s1.reshape(1, 1, 20)
    wj = jnp.arange(28)[:, None]
    w2 = jnp.arange(8)[None, :]
    mats = []
    for j in range(3):
        d = wj - (3 * w2 + j)
        valid = (d >= 0) & (d < 5)
        t = w[:, jnp.clip(d, 0, 4), :]
        t = jnp.where(valid[None, :, :, None], t, 0.0)
        mats.append(t.reshape(140, 160))
    return jnp.stack(mats)


def _band_conv2(w2r):
    w = w2r.reshape(5, 5, 20, 50)
    wj = jnp.arange(8)[:, None]
    wo = jnp.arange(4)[None, :]
    d = wj - wo
    valid = (d >= 0) & (d < 5)
    t = w[:, jnp.clip(d, 0, 4), :, :]
    t = jnp.where(valid[None, :, :, None, None], t, jnp.bfloat16(0))
    return t.transpose(0, 1, 3, 2, 4).reshape(5, 160, 200)


def kernel(x, w1r, s1, t1, w2r, s2, t2, fw1p, s3, t3, fw2t, fb2r):
    N = x.shape[0]
    bt = 256
    x = x.reshape(N, 28, 28)
    w1b = _band_conv1(w1r, s1)
    t1t = jnp.tile(t1, (1, 8))
    w2b = _band_conv2(w2r)
    s2t = jnp.tile(s2, (1, 4))
    t2t = jnp.tile(t2, (1, 4))

    fn = functools.partial(_fused_net_kernel, bt=bt)
    return pl.pallas_call(
        fn,
        out_shape=jax.ShapeDtypeStruct((N, 10), jnp.float32),
        grid=(N // bt,),
        in_specs=[
            pl.BlockSpec((bt, 28, 28), lambda n: (n, 0, 0)),
            pl.BlockSpec((3, 140, 160), lambda n: (0, 0, 0)),
            pl.BlockSpec((1, 160), lambda n: (0, 0)),
            pl.BlockSpec((5, 160, 200), lambda n: (0, 0, 0)),
            pl.BlockSpec((1, 200), lambda n: (0, 0)),
            pl.BlockSpec((1, 200), lambda n: (0, 0)),
            pl.BlockSpec((200, 500), lambda n: (0, 0)),
            pl.BlockSpec((1, 500), lambda n: (0, 0)),
            pl.BlockSpec((1, 500), lambda n: (0, 0)),
            pl.BlockSpec((500, 10), lambda n: (0, 0)),
            pl.BlockSpec((1, 10), lambda n: (0, 0)),
        ],
        out_specs=pl.BlockSpec((bt, 10), lambda n: (n, 0)),
        compiler_params=pltpu.CompilerParams(
            dimension_semantics=("parallel",)),
    )(x, w1b, t1t, w2b, s2t, t2t, fw1p, s3, t3, fw2t, fb2r)


# speedup vs baseline: 27.2122x; 1.6265x over previous
import functools

import jax
import jax.numpy as jnp
from jax.experimental import pallas as pl
from jax.experimental.pallas import tpu as pltpu


def _fused_net_kernel(x_ref, w1_ref, t1_ref, w2_ref, s2_ref, t2_ref,
                      fw1_ref, s3_ref, t3_ref, fw2_ref, fb2_ref, o_ref, *, bt):
    x = x_ref[...]
    pieces = []
    for r in range(7):
        a, b = divmod(r, 3)
        pieces.append(x[:, b, a:a + 8, :].reshape(bt * 8, 28))
    p = jnp.concatenate(pieces, axis=1)
    h = jnp.dot(p, w1_ref[0], preferred_element_type=jnp.float32)
    for m in range(1, 9):
        h = jnp.maximum(
            h, jnp.dot(p, w1_ref[m], preferred_element_type=jnp.float32))
    h = jnp.maximum(h + t1_ref[...], 0.0)
    y1 = h.astype(jnp.bfloat16).reshape(bt, 8, 160)

    acc = jnp.zeros((bt * 4, 200), jnp.float32)
    for ki in range(5):
        q = y1[:, ki:ki + 4, :].reshape(bt * 4, 160)
        acc = acc + jnp.dot(q, w2_ref[ki], preferred_element_type=jnp.float32)
    z = jnp.maximum(acc * s2_ref[...] + t2_ref[...], 0.0)
    z = jnp.concatenate(
        [jnp.maximum(z[:, 0:50], z[:, 50:100]),
         jnp.maximum(z[:, 100:150], z[:, 150:200])], axis=1)
    z = jnp.max(z.reshape(bt * 2, 2, 100), axis=1)
    z = z.reshape(bt, 2, 100)
    flat = jnp.concatenate([z[:, 0, :], z[:, 1, :]], axis=1)
    flat = flat.astype(jnp.bfloat16)

    g = jnp.dot(flat, fw1_ref[...], preferred_element_type=jnp.float32)
    g = jnp.maximum(g * s3_ref[...] + t3_ref[...], 0.0).astype(jnp.bfloat16)
    o = jnp.dot(g, fw2_ref[...], preferred_element_type=jnp.float32)
    o_ref[...] = o + fb2_ref[...]


def _band_conv1(w1r, s1):
    w = w1r.reshape(5, 5, 20) * s1.reshape(1, 1, 20)
    r = jnp.arange(7)
    wj = jnp.arange(28)[:, None]
    w2 = jnp.arange(8)[None, :]
    mats = []
    for i in range(3):
        dr = r - i
        vr = (dr >= 0) & (dr < 5)
        for j in range(3):
            dj = wj - (3 * w2 + j)
            vj = (dj >= 0) & (dj < 5)
            t = w[jnp.clip(dr, 0, 4)][:, jnp.clip(dj, 0, 4), :]
            t = jnp.where(vr[:, None, None, None] & vj[None, :, :, None],
                          t, 0.0)
            mats.append(t.reshape(196, 160))
    return jnp.stack(mats)


def _band_conv2(w2r):
    w = w2r.reshape(5, 5, 20, 50)
    wj = jnp.arange(8)[:, None]
    wo = jnp.arange(4)[None, :]
    d = wj - wo
    valid = (d >= 0) & (d < 5)
    t = w[:, jnp.clip(d, 0, 4), :, :]
    t = jnp.where(valid[None, :, :, None, None], t, jnp.bfloat16(0))
    return t.transpose(0, 1, 3, 2, 4).reshape(5, 160, 200)


def kernel(x, w1r, s1, t1, w2r, s2, t2, fw1p, s3, t3, fw2t, fb2r):
    N = x.shape[0]
    bt = 256
    xg = jnp.pad(x.reshape(N, 28, 28), ((0, 0), (0, 2), (0, 0)))
    xg = xg.reshape(N, 10, 3, 28).transpose(0, 2, 1, 3)
    w1b = _band_conv1(w1r, s1)
    t1t = jnp.tile(t1, (1, 8))
    w2b = _band_conv2(w2r)
    s2t = jnp.tile(s2, (1, 4))
    t2t = jnp.tile(t2, (1, 4))

    fn = functools.partial(_fused_net_kernel, bt=bt)
    return pl.pallas_call(
        fn,
        out_shape=jax.ShapeDtypeStruct((N, 10), jnp.float32),
        grid=(N // bt,),
        in_specs=[
            pl.BlockSpec((bt, 3, 10, 28), lambda n: (n, 0, 0, 0)),
            pl.BlockSpec((9, 196, 160), lambda n: (0, 0, 0)),
            pl.BlockSpec((1, 160), lambda n: (0, 0)),
            pl.BlockSpec((5, 160, 200), lambda n: (0, 0, 0)),
            pl.BlockSpec((1, 200), lambda n: (0, 0)),
            pl.BlockSpec((1, 200), lambda n: (0, 0)),
            pl.BlockSpec((200, 500), lambda n: (0, 0)),
            pl.BlockSpec((1, 500), lambda n: (0, 0)),
            pl.BlockSpec((1, 500), lambda n: (0, 0)),
            pl.BlockSpec((500, 10), lambda n: (0, 0)),
            pl.BlockSpec((1, 10), lambda n: (0, 0)),
        ],
        out_specs=pl.BlockSpec((bt, 10), lambda n: (n, 0)),
        compiler_params=pltpu.CompilerParams(
            dimension_semantics=("parallel",)),
    )(xg, w1b, t1t, w2b, s2t, t2t, fw1p, s3, t3, fw2t, fb2r)
